# Initial kernel scaffold; baseline (speedup 1.0000x reference)
import jax
import jax.numpy as jnp
from jax.experimental import pallas as pl


def kernel(x, edge_index, edge_attr, conditions, batch, ne_w1, ne_b1, ne_w2, ne_b2, ee_w1, ee_b1, ee_w2, ee_b2, l0_e_w1, l0_e_b1, l0_e_w2, l0_e_b2, l0_n_w1, l0_n_b1, l0_n_w2, l0_n_b2, l1_e_w1, l1_e_b1, l1_e_w2, l1_e_b2, l1_n_w1, l1_n_b1, l1_n_w2, l1_n_b2, nd_w1, nd_b1, nd_w2, nd_b2):
    raise NotImplementedError("write your pallas kernel here")


# speedup vs baseline: 3.3187x; 3.3187x over previous
import functools

import jax
import jax.numpy as jnp
from jax import lax
from jax.experimental import pallas as pl
from jax.experimental.pallas import tpu as pltpu
from jax.experimental.pallas import tpu_sc as plsc

N = 10000
E = 320000
NPAD = 10240
NBLK = 1024
NGRID = NPAD // NBLK
CH = 112
NW = 32
NCI = 90
EP = CH * NW * NCI
EBLK = 1920
EGRID = EP // EBLK
SUB_ROWS = NPAD // 16

_f32 = jnp.float32


def _dot(a, b):
    return jnp.dot(a, b, preferred_element_type=_f32)


def _wprep_body(ee_w2, ee_b2, w1e0, b1_0, e0_w2, e0_b2, w1e1, b1_1,
                wec, bec, w01, b01):
    wec[...] = _dot(ee_w2[...], w1e0[...])
    bec[...] = _dot(ee_b2[...], w1e0[...]) + b1_0[...]
    w01[...] = _dot(e0_w2[...], w1e1[...])
    b01[...] = _dot(e0_b2[...], w1e1[...]) + b1_1[...]


def _weight_prep(ee_w2, ee_b2, w1e0, b1_0, e0_w2, e0_b2, w1e1, b1_1):
    full = lambda s: pl.BlockSpec(s, lambda: (0,) * len(s))
    return pl.pallas_call(
        _wprep_body,
        grid=(),
        in_specs=[full(a.shape) for a in
                  (ee_w2, ee_b2, w1e0, b1_0, e0_w2, e0_b2, w1e1, b1_1)],
        out_specs=[full((128, 128)), full((1, 128)), full((128, 128)), full((1, 128))],
        out_shape=[jax.ShapeDtypeStruct((128, 128), _f32),
                   jax.ShapeDtypeStruct((1, 128), _f32),
                   jax.ShapeDtypeStruct((128, 128), _f32),
                   jax.ShapeDtypeStruct((1, 128), _f32)],
    )(ee_w2, ee_b2, w1e0, b1_0, e0_w2, e0_b2, w1e1, b1_1)


def _node_enc_body(xb, bf, conds, w1a, w1b, b1, w2, b2, w1s, w1d,
                   h_out, hs_out, hd_out):
    lanes = lax.broadcasted_iota(jnp.int32, (NBLK, 8), 1)
    oneh = jnp.where(bf[...] == lanes, 1.0, 0.0)
    cond = _dot(oneh, conds[...])
    pre = _dot(xb[...], w1a[...]) + _dot(cond, w1b[...]) + b1[...]
    h = _dot(jnp.maximum(pre, 0.0), w2[...]) + b2[...]
    h_out[...] = h
    hs_out[...] = _dot(h, w1s[...])
    hd_out[...] = _dot(h, w1d[...])


def _node_enc(x, bf, conds, w1a, w1b, b1, w2, b2, w1s, w1d):
    full = lambda s: pl.BlockSpec(s, lambda i: (0,) * len(s))
    nspec = pl.BlockSpec((NBLK, 128), lambda i: (i, 0))
    return pl.pallas_call(
        _node_enc_body,
        grid=(NGRID,),
        in_specs=[nspec, pl.BlockSpec((NBLK, 1), lambda i: (i, 0)),
                  full(conds.shape), full(w1a.shape), full(w1b.shape),
                  full(b1.shape), full(w2.shape), full(b2.shape),
                  full(w1s.shape), full(w1d.shape)],
        out_specs=[nspec, nspec, nspec],
        out_shape=[jax.ShapeDtypeStruct((NPAD, 128), _f32)] * 3,
    )(x, bf, conds, w1a, w1b, b1, w2, b2, w1s, w1d)


def _edge_enc_body(ea, w1, b1, wec, bec, c_out):
    a = jnp.maximum(_dot(ea[...], w1[...]) + b1[...], 0.0)
    c_out[...] = _dot(a, wec[...]) + bec[...]


def _edge_enc(edge_attr, w1, b1, wec, bec):
    full = lambda s: pl.BlockSpec(s, lambda i: (0,) * len(s))
    return pl.pallas_call(
        _edge_enc_body,
        grid=(EGRID,),
        in_specs=[pl.BlockSpec((EBLK, 16), lambda i: (i, 0)),
                  full(w1.shape), full(b1.shape), full(wec.shape), full(bec.shape)],
        out_specs=pl.BlockSpec((EBLK, 128), lambda i: (i, 0)),
        out_shape=jax.ShapeDtypeStruct((EP, 128), _f32),
    )(edge_attr, w1, b1, wec, bec)


def _mid_body(r, w, b, c_out):
    c_out[...] = _dot(r[...], w[...]) + b[...]


def _mid(r0, w01, b01):
    full = lambda s: pl.BlockSpec(s, lambda i: (0,) * len(s))
    espec = pl.BlockSpec((EBLK, 128), lambda i: (i, 0))
    return pl.pallas_call(
        _mid_body,
        grid=(EGRID,),
        in_specs=[espec, full(w01.shape), full(b01.shape)],
        out_specs=espec,
        out_shape=jax.ShapeDtypeStruct((EP, 128), _f32),
    )(r0, w01, b01)


def _node_upd_body(final, hb, Sb, Cb, e_w2, e_b2, w1a, w1b, b1, w2, b2,
                   wx1, wx2, bx1, *outs):
    S = Sb[0] + Sb[1]
    cnt = Cb[0] + Cb[1]
    agg = _dot(S, e_w2[...]) / jnp.maximum(cnt, 1.0)
    agg = agg + e_b2[...] * jnp.where(cnt > 0, 1.0, 0.0)
    pre = _dot(hb[...], w1a[...]) + _dot(agg, w1b[...]) + b1[...]
    hn = _dot(jnp.maximum(pre, 0.0), w2[...]) + b2[...] + hb[...]
    if final:
        bx2 = outs[-1]
        o = _dot(jnp.maximum(_dot(hn, wx1[...]) + bx1[...], 0.0), wx2[...]) + bx2[...]
        outs[0][...] = o
    else:
        outs[0][...] = hn
        outs[1][...] = _dot(hn, wx1[...])
        outs[2][...] = _dot(hn, wx2[...])


def _node_upd(h, S, C, e_w2, e_b2, w1a, w1b, b1, w2, b2, wx1, wx2, bx1, bx2,
              final):
    full = lambda s: pl.BlockSpec(s, lambda i: (0,) * len(s))
    nspec = pl.BlockSpec((NBLK, 128), lambda i: (i, 0))
    weights = (e_w2, e_b2, w1a, w1b, b1, w2, b2, wx1, wx2, bx1)
    in_specs = [nspec,
                pl.BlockSpec((2, NBLK, 128), lambda i: (0, i, 0)),
                pl.BlockSpec((2, NBLK, 1), lambda i: (0, i, 0))]
    in_specs += [full(w.shape) for w in weights]
    if final:
        body = functools.partial(_node_upd_body, True)
        in_specs.append(full(bx2.shape))

        def bodyf(hb, Sb, Cb, e_w2_, e_b2_, w1a_, w1b_, b1_, w2_, b2_,
                  wx1_, wx2_, bx1_, bx2_, o_ref):
            _node_upd_body(True, hb, Sb, Cb, e_w2_, e_b2_, w1a_, w1b_, b1_,
                           w2_, b2_, wx1_, wx2_, bx1_, o_ref, bx2_)
        return pl.pallas_call(
            bodyf,
            grid=(NGRID,),
            in_specs=in_specs,
            out_specs=nspec,
            out_shape=jax.ShapeDtypeStruct((NPAD, 128), _f32),
        )(h, S, C, *weights, bx2)
    else:
        body = functools.partial(_node_upd_body, False)
        return pl.pallas_call(
            body,
            grid=(NGRID,),
            in_specs=in_specs,
            out_specs=[nspec, nspec, nspec],
            out_shape=[jax.ShapeDtypeStruct((NPAD, 128), _f32)] * 3,
        )(h, S, C, *weights)


@functools.cache
def _sc_layer(first):
    mesh = plsc.VectorSubcoreMesh(core_axis_name="c", subcore_axis_name="s")
    if first:
        out_type = [jax.ShapeDtypeStruct((2, NPAD, 128), _f32),
                    jax.ShapeDtypeStruct((2, NPAD), _f32),
                    jax.ShapeDtypeStruct((EP, 128), _f32)]
    else:
        out_type = jax.ShapeDtypeStruct((2, NPAD, 128), _f32)
    scratch = [
        pltpu.VMEM((256,), jnp.int32),
        pltpu.VMEM((256,), jnp.int32),
        pltpu.VMEM((CH,), jnp.int32),
        pltpu.VMEM((CH,), jnp.int32),
        pltpu.VMEM((CH, 128), _f32),
        pltpu.VMEM((CH, 128), _f32),
        pltpu.VMEM((CH, 128), _f32),
        pltpu.VMEM((CH,), _f32),
        pltpu.VMEM((SUB_ROWS,), _f32),
        pltpu.VMEM_SHARED((NPAD, 128), _f32),
        pltpu.VMEM_SHARED((NPAD,), _f32),
        pltpu.SemaphoreType.DMA,
        pltpu.SemaphoreType.DMA,
        pltpu.SemaphoreType.DMA,
        pltpu.SemaphoreType.DMA,
        pltpu.SemaphoreType.DMA,
        pltpu.SemaphoreType.DMA,
        pltpu.SemaphoreType.DMA,
    ]

    def body(row_hbm, col_hbm, hs_hbm, hd_hbm, c_hbm, *rest):
        if first:
            S_out, C_out, r_out = rest[0], rest[1], rest[2]
            rest = rest[3:]
        else:
            S_out = rest[0]
            r_out = None
            rest = rest[1:]
        (idx01_0, idx01_1, idx_w0, idx_w1,
         rows_s, rows_d, c_v, ones_v, zb2,
         S_acc, C_acc,
         sem_pr0, sem_pr1, sem_pc0, sem_pc1,
         sem_gs, sem_gd, sem_c) = rest

        cid = lax.axis_index("c")
        sid = lax.axis_index("s")
        wid = sid * 2 + cid
        B = ((idx01_0, idx_w0, sem_pr0, sem_pc0),
             (idx01_1, idx_w1, sem_pr1, sem_pc1))

        def off_of(j):
            return (wid + j * NW) * CH

        def issue_p(j, b):
            idx01, _, sem_pr, sem_pc = B[b]
            off = off_of(j)
            pltpu.async_copy(row_hbm.at[pl.ds(off, CH)],
                             idx01.at[pl.ds(0, CH)], sem_pr)
            pltpu.async_copy(col_hbm.at[pl.ds(off, CH)],
                             idx01.at[pl.ds(128, CH)], sem_pc)

        def wait_p(b):
            idx01, _, sem_pr, sem_pc = B[b]
            pltpu.make_async_copy(row_hbm.at[pl.ds(0, CH)],
                                  idx01.at[pl.ds(0, CH)], sem_pr).wait()
            pltpu.make_async_copy(col_hbm.at[pl.ds(0, CH)],
                                  idx01.at[pl.ds(128, CH)], sem_pc).wait()

        def issue_g(b):
            idx01 = B[b][0]
            da = pltpu.async_copy(hs_hbm.at[idx01.at[pl.ds(0, CH)]],
                                 rows_s, sem_gs)
            db = pltpu.async_copy(hd_hbm.at[idx01.at[pl.ds(128, CH)]],
                                 rows_d, sem_gd)
            return da, db

        def issue_c(j):
            pltpu.async_copy(c_hbm.at[pl.ds(off_of(j), CH)], c_v, sem_c)

        def wait_c():
            pltpu.make_async_copy(c_hbm.at[pl.ds(0, CH)], c_v, sem_c).wait()

        def compute(b):
            idx01, idx_w = B[b][0], B[b][1]

            def krow(a, _):
                for u in range(8):
                    bb = u * 16
                    v = (rows_s[a, pl.ds(bb, 16)] + rows_d[a, pl.ds(bb, 16)]
                         + c_v[a, pl.ds(bb, 16)])
                    rows_s[a, pl.ds(bb, 16)] = jnp.maximum(v, 0.0)
                return 0
            lax.fori_loop(0, CH, krow, 0)
            for u in range(CH // 16):
                idx_w[pl.ds(u * 16, 16)] = idx01[pl.ds(u * 16, 16)]

        def issue_w(j, b):
            idx_w = B[b][1]
            pltpu.sync_copy(rows_s, S_acc.at[idx_w], add=True)
            if first:
                pltpu.sync_copy(ones_v, C_acc.at[idx_w], add=True)
                pltpu.sync_copy(rows_s, r_out.at[pl.ds(off_of(j), CH)])

        zero16 = jnp.zeros((16,), _f32)
        one16 = jnp.full((16,), 1.0, _f32)

        def zloop(i, _):
            c_v[i // 8, pl.ds((i % 8) * 16, 16)] = zero16
            return 0
        lax.fori_loop(0, CH * 8, zloop, 0)

        def z2loop(i, _):
            zb2[pl.ds(i * 16, 16)] = zero16
            return 0
        lax.fori_loop(0, SUB_ROWS // 16, z2loop, 0)

        def oloop(i, _):
            ones_v[pl.ds(i * 16, 16)] = one16
            return 0
        lax.fori_loop(0, CH // 16, oloop, 0)

        r0 = sid * SUB_ROWS
        zoff = 0
        while zoff < SUB_ROWS:
            zn = min(CH, SUB_ROWS - zoff)
            pltpu.sync_copy(c_v.at[pl.ds(0, zn)], S_acc.at[pl.ds(r0 + zoff, zn)])
            zoff += zn
        pltpu.sync_copy(zb2, C_acc.at[pl.ds(r0, SUB_ROWS)])
        plsc.subcore_barrier()

        def x1_iter(j, cur, nxt, has_next=True):
            wait_p(cur)
            da, db = issue_g(cur)
            if has_next:
                issue_p(j + 1, nxt)
            da.wait()
            db.wait()
            wait_c()
            compute(cur)
            if has_next:
                issue_c(j + 1)
            issue_w(j, cur)

        issue_p(0, 0)
        issue_c(0)

        def loop_body(i, _):
            j = 2 * i
            x1_iter(j, 0, 1)
            x1_iter(j + 1, 1, 0)
            return 0
        lax.fori_loop(0, NCI // 2 - 1, loop_body, 0)
        x1_iter(NCI - 2, 0, 1)
        x1_iter(NCI - 1, 1, 0, has_next=False)
        plsc.subcore_barrier()

        doff = 0
        while doff < SUB_ROWS:
            dn = min(CH, SUB_ROWS - doff)
            pltpu.sync_copy(S_acc.at[pl.ds(r0 + doff, dn)], c_v.at[pl.ds(0, dn)])
            pltpu.sync_copy(c_v.at[pl.ds(0, dn)], S_out.at[cid, pl.ds(r0 + doff, dn)])
            doff += dn
        if first:
            pltpu.sync_copy(C_acc.at[pl.ds(r0, SUB_ROWS)], zb2)
            pltpu.sync_copy(zb2, C_out.at[cid, pl.ds(r0, SUB_ROWS)])

    return pl.kernel(body, mesh=mesh, out_type=out_type, scratch_types=scratch)


@jax.jit
def kernel(x, edge_index, edge_attr, conditions, batch,
           ne_w1, ne_b1, ne_w2, ne_b2, ee_w1, ee_b1, ee_w2, ee_b2,
           l0_e_w1, l0_e_b1, l0_e_w2, l0_e_b2,
           l0_n_w1, l0_n_b1, l0_n_w2, l0_n_b2,
           l1_e_w1, l1_e_b1, l1_e_w2, l1_e_b2,
           l1_n_w1, l1_n_b1, l1_n_w2, l1_n_b2,
           nd_w1, nd_b1, nd_w2, nd_b2):
    row = jnp.pad(edge_index[0], (0, EP - E), constant_values=N)
    col = jnp.pad(edge_index[1], (0, EP - E), constant_values=N)
    ea_pad = jnp.pad(edge_attr, ((0, EP - E), (0, 0)))
    x_pad = jnp.pad(x, ((0, NPAD - N), (0, 0)))
    bf = jnp.pad(batch, (0, NPAD - N)).reshape(NPAD, 1)

    r2 = lambda b: b.reshape(1, 128)
    zeros_b = jnp.zeros((1, 128), _f32)

    wec, bec, w01, b01 = _weight_prep(
        ee_w2, r2(ee_b2), l0_e_w1[256:384], r2(l0_e_b1),
        l0_e_w2, r2(l0_e_b2), l1_e_w1[256:384], r2(l1_e_b1))

    h0, hs0, hd0 = _node_enc(
        x_pad, bf, conditions, ne_w1[0:128], ne_w1[128:144], r2(ne_b1),
        ne_w2, r2(ne_b2), l0_e_w1[0:128], l0_e_w1[128:256])

    c0 = _edge_enc(ea_pad, ee_w1, ee_b1.reshape(1, 128), wec, bec)

    S0, C, r0 = _sc_layer(True)(row, col, hs0, hd0, c0)
    C3 = C.reshape(2, NPAD, 1)

    c1 = _mid(r0, w01, b01)

    h1, hs1, hd1 = _node_upd(
        h0, S0, C3, l0_e_w2, r2(l0_e_b2),
        l0_n_w1[0:128], l0_n_w1[128:256], r2(l0_n_b1), l0_n_w2, r2(l0_n_b2),
        l1_e_w1[0:128], l1_e_w1[128:256], zeros_b, zeros_b, final=False)

    S1 = _sc_layer(False)(row, col, hs1, hd1, c1)

    out = _node_upd(
        h1, S1, C3, l1_e_w2, r2(l1_e_b2),
        l1_n_w1[0:128], l1_n_w1[128:256], r2(l1_n_b1), l1_n_w2, r2(l1_n_b2),
        nd_w1, nd_w2, r2(nd_b1), r2(nd_b2), final=True)

    return out[:N]
